# speedup vs baseline: 1.4823x; 1.4823x over previous
import functools

import jax
import jax.numpy as jnp
from jax import lax
from jax.experimental import pallas as pl
from jax.experimental.pallas import tpu as pltpu
from jax.experimental.pallas import tpu_sc as plsc

_INFO = plsc.get_sparse_core_info()
_NC = _INFO.num_cores
_NS = _INFO.num_subcores
_NW = _NC * _NS

_CHUNK = 64


def _make_gather(V, D, B):
  assert B % _NW == 0
  b_per_w = B // _NW
  assert b_per_w % _CHUNK == 0
  n_chunks = b_per_w // _CHUNK
  mesh = plsc.VectorSubcoreMesh(core_axis_name="c", subcore_axis_name="s")

  @functools.partial(
      pl.kernel,
      mesh=mesh,
      out_type=jax.ShapeDtypeStruct((B, D), jnp.float32),
      scratch_types=[
          pltpu.VMEM((b_per_w,), jnp.int32),
          pltpu.VMEM((_CHUNK, D), jnp.float32),
          pltpu.VMEM((_CHUNK, D), jnp.float32),
          pltpu.SemaphoreType.DMA,
          pltpu.SemaphoreType.DMA,
          pltpu.SemaphoreType.DMA,
          pltpu.SemaphoreType.DMA,
      ],
  )
  def gather_kernel(idx_hbm, table_hbm, out_hbm, idx_v, rows0, rows1,
                    gsem0, gsem1, osem0, osem1):
    wid = lax.axis_index("s") * _NC + lax.axis_index("c")
    base = wid * b_per_w
    pltpu.sync_copy(idx_hbm.at[pl.ds(base, b_per_w)], idx_v)

    bufs = (rows0, rows1)
    gsems = (gsem0, gsem1)
    osems = (osem0, osem1)
    gather = [None, None]
    put = [None, None]

    gather[0] = pltpu.async_copy(
        table_hbm.at[idx_v.at[pl.ds(0, _CHUNK)]], bufs[0], gsems[0])
    for c in range(n_chunks):
      b = c % 2
      nb = (c + 1) % 2
      if c + 1 < n_chunks:
        if put[nb] is not None:
          put[nb].wait()
        gather[nb] = pltpu.async_copy(
            table_hbm.at[idx_v.at[pl.ds((c + 1) * _CHUNK, _CHUNK)]],
            bufs[nb], gsems[nb])
      gather[b].wait()
      put[b] = pltpu.async_copy(
          bufs[b], out_hbm.at[pl.ds(base + c * _CHUNK, _CHUNK)], osems[b])
    for b in range(2):
      if put[b] is not None:
        put[b].wait()

  return gather_kernel


def kernel(tokens, W_E):
  n_rows, n_cols = tokens.shape
  V, D = W_E.shape
  B = n_rows * n_cols
  idx = tokens.reshape(B).astype(jnp.int32)
  out = _make_gather(V, D, B)(idx, W_E)
  return out.reshape(n_rows, n_cols, D)


# speedup vs baseline: 1.4827x; 1.0002x over previous
import functools

import jax
import jax.numpy as jnp
from jax import lax
from jax.experimental import pallas as pl
from jax.experimental.pallas import tpu as pltpu
from jax.experimental.pallas import tpu_sc as plsc

_INFO = plsc.get_sparse_core_info()
_NC = _INFO.num_cores
_NS = _INFO.num_subcores
_NW = _NC * _NS

_CHUNK = 64


def _make_gather(V, D, n_rows, n_cols):
  B = n_rows * n_cols
  assert B % _NW == 0
  b_per_w = B // _NW
  assert b_per_w % _CHUNK == 0 and n_cols % b_per_w == 0
  w_per_row = n_cols // b_per_w
  n_chunks = b_per_w // _CHUNK
  mesh = plsc.VectorSubcoreMesh(core_axis_name="c", subcore_axis_name="s")

  @functools.partial(
      pl.kernel,
      mesh=mesh,
      out_type=jax.ShapeDtypeStruct((n_rows, n_cols, D), jnp.float32),
      scratch_types=[
          pltpu.VMEM((b_per_w,), jnp.int32),
          pltpu.VMEM((_CHUNK, D), jnp.float32),
          pltpu.VMEM((_CHUNK, D), jnp.float32),
          pltpu.SemaphoreType.DMA,
          pltpu.SemaphoreType.DMA,
          pltpu.SemaphoreType.DMA,
          pltpu.SemaphoreType.DMA,
      ],
  )
  def gather_kernel(idx_hbm, table_hbm, out_hbm, idx_v, rows0, rows1,
                    gsem0, gsem1, osem0, osem1):
    wid = lax.axis_index("s") * _NC + lax.axis_index("c")
    row = wid // w_per_row
    col0 = (wid % w_per_row) * b_per_w
    pltpu.sync_copy(idx_hbm.at[row, pl.ds(col0, b_per_w)], idx_v)

    bufs = (rows0, rows1)
    gsems = (gsem0, gsem1)
    osems = (osem0, osem1)
    gather = [None, None]
    put = [None, None]

    gather[0] = pltpu.async_copy(
        table_hbm.at[idx_v.at[pl.ds(0, _CHUNK)]], bufs[0], gsems[0])
    for c in range(n_chunks):
      b = c % 2
      nb = (c + 1) % 2
      if c + 1 < n_chunks:
        if put[nb] is not None:
          put[nb].wait()
        gather[nb] = pltpu.async_copy(
            table_hbm.at[idx_v.at[pl.ds((c + 1) * _CHUNK, _CHUNK)]],
            bufs[nb], gsems[nb])
      gather[b].wait()
      put[b] = pltpu.async_copy(
          bufs[b], out_hbm.at[row, pl.ds(col0 + c * _CHUNK, _CHUNK)],
          osems[b])
    for b in range(2):
      if put[b] is not None:
        put[b].wait()

  return gather_kernel


def kernel(tokens, W_E):
  n_rows, n_cols = tokens.shape
  V, D = W_E.shape
  idx = tokens.astype(jnp.int32)
  return _make_gather(V, D, n_rows, n_cols)(idx, W_E)


# speedup vs baseline: 1.5238x; 1.0277x over previous
import functools

import jax
import jax.numpy as jnp
from jax import lax
from jax.experimental import pallas as pl
from jax.experimental.pallas import tpu as pltpu
from jax.experimental.pallas import tpu_sc as plsc

_INFO = plsc.get_sparse_core_info()
_NC = _INFO.num_cores
_NS = _INFO.num_subcores
_NW = _NC * _NS

_CHUNK = 32
_NBUF = 4


def _make_gather(V, D, n_rows, n_cols):
  B = n_rows * n_cols
  assert B % _NW == 0
  b_per_w = B // _NW
  assert b_per_w % _CHUNK == 0 and n_cols % b_per_w == 0
  w_per_row = n_cols // b_per_w
  n_chunks = b_per_w // _CHUNK
  mesh = plsc.VectorSubcoreMesh(core_axis_name="c", subcore_axis_name="s")

  @functools.partial(
      pl.kernel,
      mesh=mesh,
      out_type=jax.ShapeDtypeStruct((n_rows, n_cols, D), jnp.float32),
      scratch_types=(
          [pltpu.VMEM((b_per_w,), jnp.int32)]
          + [pltpu.VMEM((_CHUNK, D), jnp.float32) for _ in range(_NBUF)]
          + [pltpu.SemaphoreType.DMA for _ in range(2 * _NBUF)]
      ),
  )
  def gather_kernel(idx_hbm, table_hbm, out_hbm, idx_v, *scratch):
    bufs = scratch[:_NBUF]
    gsems = scratch[_NBUF:2 * _NBUF]
    osems = scratch[2 * _NBUF:]
    wid = lax.axis_index("s") * _NC + lax.axis_index("c")
    row = wid // w_per_row
    col0 = (wid % w_per_row) * b_per_w
    pltpu.sync_copy(idx_hbm.at[row, pl.ds(col0, b_per_w)], idx_v)

    gather = [None] * _NBUF
    put = [None] * _NBUF

    for c in range(min(_NBUF, n_chunks)):
      gather[c] = pltpu.async_copy(
          table_hbm.at[idx_v.at[pl.ds(c * _CHUNK, _CHUNK)]], bufs[c],
          gsems[c])
    for c in range(n_chunks):
      b = c % _NBUF
      gather[b].wait()
      put[b] = pltpu.async_copy(
          bufs[b], out_hbm.at[row, pl.ds(col0 + c * _CHUNK, _CHUNK)],
          osems[b])
      nxt = c + _NBUF
      if nxt < n_chunks:
        put[b].wait()
        gather[b] = pltpu.async_copy(
            table_hbm.at[idx_v.at[pl.ds(nxt * _CHUNK, _CHUNK)]], bufs[b],
            gsems[b])
    for b in range(_NBUF):
      if put[b] is not None:
        put[b].wait()

  return gather_kernel


def kernel(tokens, W_E):
  n_rows, n_cols = tokens.shape
  V, D = W_E.shape
  idx = tokens.astype(jnp.int32)
  return _make_gather(V, D, n_rows, n_cols)(idx, W_E)
